# baseline (speedup 1.0000x reference)
import functools

import jax
import jax.numpy as jnp
from jax import lax
from jax.experimental import pallas as pl
from jax.experimental.pallas import tpu as pltpu
from jax.experimental.pallas import tpu_sc as plsc

_C = 128
_NBUF = 4
_OBUF = 2


@functools.lru_cache(maxsize=None)
def _build(N, D, HB):
    info = plsc.get_sparse_core_info()
    nw = info.num_cores * info.num_subcores
    nper = N // nw
    steps = nper // _C
    outer = steps // _NBUF
    nfa = D // 8
    bblocks = HB // _C
    mesh = plsc.VectorSubcoreMesh(core_axis_name="c", subcore_axis_name="s")

    def body(t1, i1, t2, i2, out, i1v, i2v, r1, r2, ob, *sems):
        sg1 = sems[:_NBUF]
        sg2 = sems[_NBUF:2 * _NBUF]
        so = sems[2 * _NBUF:]
        wid = lax.axis_index("s") * info.num_cores + lax.axis_index("c")
        base = wid * nper
        iota = jax.lax.iota(jnp.int32, 16)
        dconst = []
        for f0 in (0, 16):
            f = f0 + iota
            dconst.append((f >> 3) * (8 * _C) + (f & 7) * _C)
        pltpu.sync_copy(i1.at[pl.ds(base, nper)], i1v)
        pltpu.sync_copy(i2.at[pl.ds(base, nper)], i2v)

        def issue(g, b):
            o = g * _C
            pltpu.async_copy(t1.at[i1v.at[pl.ds(o, _C)]], r1.at[b], sg1[b])
            pltpu.async_copy(t2.at[i2v.at[pl.ds(o, _C)]], r2.at[b], sg2[b])

        def wait_gather(b):
            pltpu.make_async_copy(
                t1.at[i1v.at[pl.ds(0, _C)]], r1.at[b], sg1[b]).wait()
            pltpu.make_async_copy(
                t2.at[i2v.at[pl.ds(0, _C)]], r2.at[b], sg2[b]).wait()

        def wait_scatter(b2):
            for fa in range(nfa):
                pltpu.make_async_copy(
                    ob.at[b2, pl.ds(0, 8 * _C)], out.at[pl.ds(0, 8 * _C)],
                    so[b2]).wait()

        for b in range(_NBUF):
            issue(b, b)

        def outer_step(g2, carry):
            for b in range(_NBUF):
                g = g2 * _NBUF + b
                s = wid * steps + g
                h = s // bblocks
                ba = s % bblocks
                wait_gather(b)
                b2 = b % _OBUF
                if b < _OBUF:
                    @pl.when(g2 > 0)
                    def _():
                        wait_scatter(b2)
                else:
                    wait_scatter(b2)

                def tr_body(bb, c):
                    for f0 in (0, 16):
                        sl = pl.ds(f0, 16)
                        v = r1[b, bb, sl] + r2[b, bb, sl]
                        plsc.store_scatter(ob.at[b2], [dconst[f0 // 16] + bb], v)
                    return c

                lax.fori_loop(0, _C, tr_body, 0, unroll=8)

                @pl.when(g2 < outer - 1)
                def _():
                    issue(g + _NBUF, b)

                for fa in range(nfa):
                    off = (h * nfa + fa) * (bblocks * 8 * _C) + ba * (8 * _C)
                    pltpu.async_copy(ob.at[b2, pl.ds(fa * 8 * _C, 8 * _C)],
                                     out.at[pl.ds(off, 8 * _C)], so[b2])
            return carry

        lax.fori_loop(0, outer, outer_step, 0)
        for b2 in range(_OBUF):
            wait_scatter(b2)

    return pl.kernel(
        body,
        mesh=mesh,
        out_type=jax.ShapeDtypeStruct((N * D,), jnp.float32),
        scratch_types=[
            pltpu.VMEM((nper,), jnp.int32),
            pltpu.VMEM((nper,), jnp.int32),
            pltpu.VMEM((_NBUF, _C, D), jnp.float32),
            pltpu.VMEM((_NBUF, _C, D), jnp.float32),
            pltpu.VMEM((_OBUF, nfa * 8 * _C), jnp.float32),
        ] + [pltpu.SemaphoreType.DMA] * (2 * _NBUF + _OBUF),
        compiler_params=pltpu.CompilerParams(use_tc_tiling_on_sc=False,
                                             needs_layout_passes=False),
    )


@functools.lru_cache(maxsize=None)
def _build_transpose(V, D):
    info = plsc.get_sparse_core_info()
    ncore, nsub = info.num_cores, info.num_subcores
    nw = ncore * nsub
    full_cols = V // _C
    tail_rows = V - full_cols * _C
    percol = _C * D
    ntr = D // 8
    ncols_lo = full_cols // nw
    nextra = full_cols - ncols_lo * nw
    mesh = plsc.VectorSubcoreMesh(core_axis_name="c", subcore_axis_name="s")

    def body(t1t, t2t, tail1, tail2, o1, o2, bin_, bout0, bout1, tbuf, *sems):
        bout = (bout0, bout1)
        si = sems[:2]
        so = sems[2:]
        wid = lax.axis_index("s") * ncore + lax.axis_index("c")
        base_c = wid * ncols_lo + jnp.minimum(wid, nextra)
        iota32 = jax.lax.iota(jnp.int32, 16) * D

        def issue(tbl, c, b):
            for tr in range(ntr):
                pltpu.async_copy(
                    tbl.at[pl.ds(8 * tr, 8), pl.ds(c * _C, _C)],
                    bin_.at[b, tr], si[b])

        def wait_in(tbl, b):
            for tr in range(ntr):
                pltpu.make_async_copy(
                    tbl.at[pl.ds(0, 8), pl.ds(0, _C)], bin_.at[b, tr],
                    si[b]).wait()

        def wait_out(out, b):
            pltpu.make_async_copy(
                bout[b], out.at[pl.ds(0, percol)], so[b]).wait()

        def transpose_col(b):
            for tr in range(ntr):
                for fl in range(8):
                    f = 8 * tr + fl
                    for k in range(8):
                        v = bin_[b, tr, fl, pl.ds(16 * k, 16)]
                        plsc.store_scatter(
                            bout[b], [iota32 + (16 * k * D + f)], v)

        for tbl, out in ((t1t, o1), (t2t, o2)):
            def issue2(t, b, tbl=tbl):
                issue(tbl, base_c + t, b)

            for b in range(2):
                issue2(b, b)

            def step(t2_, carry, tbl=tbl, out=out):
                for b in range(2):
                    t = t2_ * 2 + b
                    wait_in(tbl, b)
                    @pl.when(t2_ > 0)
                    def _():
                        wait_out(out, b)
                    transpose_col(b)
                    pltpu.async_copy(
                        bout[b],
                        out.at[pl.ds((base_c + t) * percol, percol)], so[b])
                    @pl.when(t + 2 < ncols_lo)
                    def _():
                        issue2(t + 2, b)
                return carry

            lax.fori_loop(0, ncols_lo // 2, step, 0)
            for b in range(2):
                wait_out(out, b)

            @pl.when(wid < nextra)
            def _(tbl=tbl, out=out):
                c0 = base_c + ncols_lo
                issue(tbl, c0, 0)
                wait_in(tbl, 0)
                transpose_col(0)
                pltpu.sync_copy(bout[0], out.at[pl.ds(c0 * percol, percol)])

        for w_own, tail, out in ((nw - 2, tail1, o1), (nw - 1, tail2, o2)):
            @pl.when(wid == w_own)
            def _(tail=tail, out=out):
                for k in range(tail_rows // 8):
                    pltpu.sync_copy(tail.at[pl.ds(8 * k, 8), :], tbuf)
                    for r in range(8):
                        for f0 in (0, 16):
                            bout0[pl.ds(r * D + f0, 16)] = tbuf[r, pl.ds(f0, 16)]
                    pltpu.sync_copy(
                        bout0.at[pl.ds(0, 8 * D)],
                        out.at[pl.ds((full_cols * _C + 8 * k) * D, 8 * D)])

    return pl.kernel(
        body,
        mesh=mesh,
        out_type=(jax.ShapeDtypeStruct((V * D,), jnp.float32),
                  jax.ShapeDtypeStruct((V * D,), jnp.float32)),
        scratch_types=[
            pltpu.VMEM((2, ntr, 8, _C), jnp.float32),
            pltpu.VMEM((percol,), jnp.float32),
            pltpu.VMEM((percol,), jnp.float32),
            pltpu.VMEM((8, _C), jnp.float32),
        ] + [pltpu.SemaphoreType.DMA] * 4,
        compiler_params=pltpu.CompilerParams(needs_layout_passes=False),
    )


def kernel(input, another_input, table1, table2):
    B, H = input.shape
    V, D = table1.shape
    N = B * H
    full_cols = V // _C
    i1 = input.T.reshape(N).astype(jnp.int32)
    i2 = another_input.T.reshape(N).astype(jnp.int32)
    tail1 = jnp.pad(table1[full_cols * _C:, :], ((0, 0), (0, _C - D)))
    tail2 = jnp.pad(table2[full_cols * _C:, :], ((0, 0), (0, _C - D)))
    o1, o2 = _build_transpose(V, D)(table1.T, table2.T, tail1, tail2)
    t1 = o1.reshape(V, D)
    t2 = o2.reshape(V, D)
    flat = _build(N, D, B)(t1, i1, t2, i2)
    out5 = flat.reshape(H, D // 8, B // 128, 8, 128)
    return out5.transpose(2, 4, 0, 1, 3).reshape(B, H, D)


# speedup vs baseline: 1.0191x; 1.0191x over previous
import functools

import jax
import jax.numpy as jnp
from jax import lax
from jax.experimental import pallas as pl
from jax.experimental.pallas import tpu as pltpu
from jax.experimental.pallas import tpu_sc as plsc

_C = 128
_NBUF = 4
_OBUF = 2


@functools.lru_cache(maxsize=None)
def _build(N, D, HB):
    info = plsc.get_sparse_core_info()
    nw = info.num_cores * info.num_subcores
    nper = N // nw
    steps = nper // _C
    outer = steps // _NBUF
    nfa = D // 8
    bblocks = HB // _C
    mesh = plsc.VectorSubcoreMesh(core_axis_name="c", subcore_axis_name="s")

    def body(t1, i1, t2, i2, out, i1v, i2v, r1, r2, ob, *sems):
        sg1 = sems[:_NBUF]
        sg2 = sems[_NBUF:2 * _NBUF]
        so = sems[2 * _NBUF:]
        wid = lax.axis_index("s") * info.num_cores + lax.axis_index("c")
        base = wid * nper
        iota = jax.lax.iota(jnp.int32, 16)
        pltpu.sync_copy(i1.at[pl.ds(base, nper)], i1v)
        pltpu.sync_copy(i2.at[pl.ds(base, nper)], i2v)

        def issue(g, b):
            o = g * _C
            pltpu.async_copy(t1.at[i1v.at[pl.ds(o, _C)]], r1.at[b], sg1[b])
            pltpu.async_copy(t2.at[i2v.at[pl.ds(o, _C)]], r2.at[b], sg2[b])

        def wait_gather(b):
            pltpu.make_async_copy(
                t1.at[i1v.at[pl.ds(0, _C)]], r1.at[b], sg1[b]).wait()
            pltpu.make_async_copy(
                t2.at[i2v.at[pl.ds(0, _C)]], r2.at[b], sg2[b]).wait()

        def wait_scatter(b2):
            for fa in range(nfa):
                pltpu.make_async_copy(
                    ob.at[b2, pl.ds(0, 8), pl.ds(0, _C)],
                    out.at[pl.ds(0, 8), :], so[b2]).wait()

        for b in range(_NBUF):
            issue(b, b)

        def outer_step(g2, carry):
            for b in range(_NBUF):
                g = g2 * _NBUF + b
                s = wid * steps + g
                h = s // bblocks
                ba = s % bblocks
                wait_gather(b)
                b2 = b % _OBUF
                if b < _OBUF:
                    @pl.when(g2 > 0)
                    def _():
                        wait_scatter(b2)
                else:
                    wait_scatter(b2)

                def tr_body(bb, c):
                    bs = jnp.full((16,), bb, jnp.int32)
                    for f0 in (0, 16):
                        sl = pl.ds(f0, 16)
                        v = r1[b, bb, sl] + r2[b, bb, sl]
                        plsc.store_scatter(ob.at[b2], [f0 + iota, bs], v)
                    return c

                lax.fori_loop(0, _C, tr_body, 0, unroll=8)

                @pl.when(g2 < outer - 1)
                def _():
                    issue(g + _NBUF, b)

                for fa in range(nfa):
                    off8 = (h * nfa + fa) * (bblocks * 8) + ba * 8
                    pltpu.async_copy(ob.at[b2, pl.ds(fa * 8, 8), pl.ds(0, _C)],
                                     out.at[pl.ds(off8, 8), :], so[b2])
            return carry

        lax.fori_loop(0, outer, outer_step, 0)
        for b2 in range(_OBUF):
            wait_scatter(b2)

    return pl.kernel(
        body,
        mesh=mesh,
        out_type=jax.ShapeDtypeStruct((N * D // _C, _C), jnp.float32),
        scratch_types=[
            pltpu.VMEM((nper,), jnp.int32),
            pltpu.VMEM((nper,), jnp.int32),
            pltpu.VMEM((_NBUF, _C, D), jnp.float32),
            pltpu.VMEM((_NBUF, _C, D), jnp.float32),
            pltpu.VMEM((_OBUF, nfa * 8, _C + 1), jnp.float32),
        ] + [pltpu.SemaphoreType.DMA] * (2 * _NBUF + _OBUF),
        compiler_params=pltpu.CompilerParams(use_tc_tiling_on_sc=False,
                                             needs_layout_passes=False),
    )


@functools.lru_cache(maxsize=None)
def _build_transpose(V, D):
    info = plsc.get_sparse_core_info()
    ncore, nsub = info.num_cores, info.num_subcores
    nw = ncore * nsub
    full_cols = V // _C
    tail_rows = V - full_cols * _C
    percol = _C * D
    ntr = D // 8
    ncols_lo = full_cols // nw
    nextra = full_cols - ncols_lo * nw
    mesh = plsc.VectorSubcoreMesh(core_axis_name="c", subcore_axis_name="s")

    def body(t1t, t2t, tail1, tail2, o1, o2, bin0, bin1, bout0, bout1, tbuf, *sems):
        binb = (bin0, bin1)
        bout = (bout0, bout1)
        si = sems[:2]
        so = sems[2:]
        wid = lax.axis_index("s") * ncore + lax.axis_index("c")
        base_c = wid * ncols_lo + jnp.minimum(wid, nextra)

        def issue(tbl, c, b):
            for tr in range(ntr):
                pltpu.async_copy(
                    tbl.at[pl.ds(8 * tr, 8), pl.ds(c * _C, _C)],
                    binb[b].at[tr, :, pl.ds(0, _C)], si[b])

        def wait_in(tbl, b):
            for tr in range(ntr):
                pltpu.make_async_copy(
                    tbl.at[pl.ds(0, 8), pl.ds(0, _C)],
                    binb[b].at[tr, :, pl.ds(0, _C)], si[b]).wait()

        def wait_out(out, b):
            pltpu.make_async_copy(
                bout[b], out.at[pl.ds(0, percol)], so[b]).wait()

        trv = jax.lax.iota(jnp.int32, 16) >> 3
        flv = jax.lax.iota(jnp.int32, 16) & 7

        def transpose_col(b):
            def row_body(r, c):
                rs = jnp.full((16,), r, jnp.int32)
                for f0 in (0, 16):
                    v = plsc.load_gather(
                        binb[b], [(f0 >> 3) + trv, flv, rs])
                    bout[b][pl.ds(r * D + f0, 16)] = v
                return c

            lax.fori_loop(0, _C, row_body, 0, unroll=8)

        for tbl, out in ((t1t, o1), (t2t, o2)):
            def issue2(t, b, tbl=tbl):
                issue(tbl, base_c + t, b)

            for b in range(2):
                issue2(b, b)

            def step(t2_, carry, tbl=tbl, out=out):
                for b in range(2):
                    t = t2_ * 2 + b
                    wait_in(tbl, b)
                    @pl.when(t2_ > 0)
                    def _():
                        wait_out(out, b)
                    transpose_col(b)
                    pltpu.async_copy(
                        bout[b],
                        out.at[pl.ds((base_c + t) * percol, percol)], so[b])
                    @pl.when(t + 2 < ncols_lo)
                    def _():
                        issue2(t + 2, b)
                return carry

            lax.fori_loop(0, ncols_lo // 2, step, 0)
            for b in range(2):
                wait_out(out, b)

            @pl.when(wid < nextra)
            def _(tbl=tbl, out=out):
                c0 = base_c + ncols_lo
                issue(tbl, c0, 0)
                wait_in(tbl, 0)
                transpose_col(0)
                pltpu.sync_copy(bout[0], out.at[pl.ds(c0 * percol, percol)])

        for w_own, tail, out in ((nw - 2, tail1, o1), (nw - 1, tail2, o2)):
            @pl.when(wid == w_own)
            def _(tail=tail, out=out):
                for k in range(tail_rows // 8):
                    pltpu.sync_copy(tail.at[pl.ds(8 * k, 8), :], tbuf)
                    for r in range(8):
                        for f0 in (0, 16):
                            bout0[pl.ds(r * D + f0, 16)] = tbuf[r, pl.ds(f0, 16)]
                    pltpu.sync_copy(
                        bout0.at[pl.ds(0, 8 * D)],
                        out.at[pl.ds((full_cols * _C + 8 * k) * D, 8 * D)])

    return pl.kernel(
        body,
        mesh=mesh,
        out_type=(jax.ShapeDtypeStruct((V * D,), jnp.float32),
                  jax.ShapeDtypeStruct((V * D,), jnp.float32)),
        scratch_types=[
            pltpu.VMEM((ntr, 8, _C + 1), jnp.float32),
            pltpu.VMEM((ntr, 8, _C + 1), jnp.float32),
            pltpu.VMEM((percol,), jnp.float32),
            pltpu.VMEM((percol,), jnp.float32),
            pltpu.VMEM((8, _C), jnp.float32),
        ] + [pltpu.SemaphoreType.DMA] * 4,
        compiler_params=pltpu.CompilerParams(needs_layout_passes=False),
    )


def kernel(input, another_input, table1, table2):
    B, H = input.shape
    V, D = table1.shape
    N = B * H
    full_cols = V // _C
    i1 = input.T.reshape(N).astype(jnp.int32)
    i2 = another_input.T.reshape(N).astype(jnp.int32)
    tail1 = jnp.pad(table1[full_cols * _C:, :], ((0, 0), (0, _C - D)))
    tail2 = jnp.pad(table2[full_cols * _C:, :], ((0, 0), (0, _C - D)))
    o1, o2 = _build_transpose(V, D)(table1.T, table2.T, tail1, tail2)
    t1 = o1.reshape(V, D)
    t2 = o2.reshape(V, D)
    flat = _build(N, D, B)(t1, i1, t2, i2)
    out5 = flat.reshape(H, D // 8, B // 128, 8, 128)
    return out5.transpose(2, 4, 0, 1, 3).reshape(B, H, D)


# speedup vs baseline: 1.0213x; 1.0022x over previous
import functools

import jax
import jax.numpy as jnp
from jax import lax
from jax.experimental import pallas as pl
from jax.experimental.pallas import tpu as pltpu
from jax.experimental.pallas import tpu_sc as plsc

_C = 128
_NBUF = 4
_OBUF = 2


@functools.lru_cache(maxsize=None)
def _build(N, D, HB):
    info = plsc.get_sparse_core_info()
    nw = info.num_cores * info.num_subcores
    nper = N // nw
    steps = nper // _C
    outer = steps // _NBUF
    nfa = D // 8
    bblocks = HB // _C
    mesh = plsc.VectorSubcoreMesh(core_axis_name="c", subcore_axis_name="s")

    def body(t1, i1, t2, i2, out, i1v, i2v, r1, r2, ob, *sems):
        sg1 = sems[:_NBUF]
        sg2 = sems[_NBUF:2 * _NBUF]
        so = sems[2 * _NBUF:]
        wid = lax.axis_index("s") * info.num_cores + lax.axis_index("c")
        base = wid * nper
        iota = jax.lax.iota(jnp.int32, 16)
        pltpu.sync_copy(i1.at[pl.ds(base, nper)], i1v)
        pltpu.sync_copy(i2.at[pl.ds(base, nper)], i2v)

        def issue(g, b):
            o = g * _C
            pltpu.async_copy(t1.at[i1v.at[pl.ds(o, _C)]], r1.at[b], sg1[b])
            pltpu.async_copy(t2.at[i2v.at[pl.ds(o, _C)]], r2.at[b], sg2[b])

        def wait_gather(b):
            pltpu.make_async_copy(
                t1.at[i1v.at[pl.ds(0, _C)]], r1.at[b], sg1[b]).wait()
            pltpu.make_async_copy(
                t2.at[i2v.at[pl.ds(0, _C)]], r2.at[b], sg2[b]).wait()

        def wait_scatter(b2):
            for fa in range(nfa):
                pltpu.make_async_copy(
                    ob.at[b2, pl.ds(0, 8), pl.ds(0, _C)],
                    out.at[pl.ds(0, 8), :], so[b2]).wait()

        for b in range(_NBUF):
            issue(b, b)

        def outer_step(g2, carry):
            for b in range(_NBUF):
                g = g2 * _NBUF + b
                s = wid * steps + g
                h = s // bblocks
                ba = s % bblocks
                wait_gather(b)
                b2 = b % _OBUF
                if b < _OBUF:
                    @pl.when(g2 > 0)
                    def _():
                        wait_scatter(b2)
                else:
                    wait_scatter(b2)

                def tr_body(bb, c):
                    bs = jnp.full((16,), bb, jnp.int32)
                    for f0 in (0, 16):
                        sl = pl.ds(f0, 16)
                        v = r1[b, bb, sl] + r2[b, bb, sl]
                        plsc.store_scatter(ob.at[b2], [f0 + iota, bs], v)
                    return c

                lax.fori_loop(0, _C, tr_body, 0, unroll=8)

                @pl.when(g2 < outer - 1)
                def _():
                    issue(g + _NBUF, b)

                for fa in range(nfa):
                    off8 = (h * nfa + fa) * (bblocks * 8) + ba * 8
                    pltpu.async_copy(ob.at[b2, pl.ds(fa * 8, 8), pl.ds(0, _C)],
                                     out.at[pl.ds(off8, 8), :], so[b2])
            return carry

        lax.fori_loop(0, outer, outer_step, 0)
        for b2 in range(_OBUF):
            wait_scatter(b2)

    return pl.kernel(
        body,
        mesh=mesh,
        out_type=jax.ShapeDtypeStruct((N * D // _C, _C), jnp.float32),
        scratch_types=[
            pltpu.VMEM((nper,), jnp.int32),
            pltpu.VMEM((nper,), jnp.int32),
            pltpu.VMEM((_NBUF, _C, D), jnp.float32),
            pltpu.VMEM((_NBUF, _C, D), jnp.float32),
            pltpu.VMEM((_OBUF, nfa * 8, _C + 1), jnp.float32),
        ] + [pltpu.SemaphoreType.DMA] * (2 * _NBUF + _OBUF),
        compiler_params=pltpu.CompilerParams(use_tc_tiling_on_sc=False,
                                             needs_layout_passes=False),
    )


@functools.lru_cache(maxsize=None)
def _build_transpose(V, D):
    info = plsc.get_sparse_core_info()
    ncore, nsub = info.num_cores, info.num_subcores
    nw = ncore * nsub
    full_cols = V // _C
    tail_rows = V - full_cols * _C
    percol = _C * D
    ncols_lo = full_cols // nw
    nextra = full_cols - ncols_lo * nw
    K = 4
    W = K * _C
    P = W + 5
    nblk = ncols_lo // K
    mesh = plsc.VectorSubcoreMesh(core_axis_name="c", subcore_axis_name="s")

    def body(t1t, t2t, tail1, tail2, o1, o2, bin0, bin1, bout0, bout1, tbuf,
             *sems):
        binb = (bin0, bin1)
        bout = (bout0, bout1)
        si = sems[:2]
        so = sems[2:]
        wid = lax.axis_index("s") * ncore + lax.axis_index("c")
        base_c = wid * ncols_lo + jnp.minimum(wid, nextra)
        iota = jax.lax.iota(jnp.int32, 16)

        def issue(tbl, t, b, width=W):
            pltpu.async_copy(
                tbl.at[:, pl.ds((base_c + t * K) * _C, width)],
                binb[b].at[:, pl.ds(0, width)], si[b])

        def wait_in(tbl, b, width=W):
            pltpu.make_async_copy(
                tbl.at[:, pl.ds(0, width)], binb[b].at[:, pl.ds(0, width)],
                si[b]).wait()

        def wait_out(out, b, words=W * D):
            pltpu.make_async_copy(
                bout[b].at[pl.ds(0, words)], out.at[pl.ds(0, words)],
                so[b]).wait()

        def transpose_strip(b, width):
            def row_body(r, c):
                rs = jnp.full((16,), r, jnp.int32)
                for f0 in (0, 16):
                    v = plsc.load_gather(binb[b], [f0 + iota, rs])
                    bout[b][pl.ds(r * D + f0, 16)] = v
                return c

            lax.fori_loop(0, width, row_body, 0, unroll=8)

        for tbl, out in ((t1t, o1), (t2t, o2)):
            def issue2(t, b, tbl=tbl):
                issue(tbl, t, b)

            for b in range(2):
                issue2(b, b)

            def step(t2_, carry, tbl=tbl, out=out):
                for b in range(2):
                    t = t2_ * 2 + b
                    wait_in(tbl, b)
                    @pl.when(t2_ > 0)
                    def _():
                        wait_out(out, b)
                    transpose_strip(b, W)
                    pltpu.async_copy(
                        bout[b].at[pl.ds(0, W * D)],
                        out.at[pl.ds((base_c + t * K) * percol, W * D)], so[b])
                    @pl.when(t + 2 < nblk)
                    def _():
                        issue2(t + 2, b)
                return carry

            lax.fori_loop(0, nblk // 2, step, 0)

            if nblk % 2:
                t = nblk - 1
                wait_in(tbl, 0)
                wait_out(out, 0)
                transpose_strip(0, W)
                pltpu.async_copy(
                    bout[0].at[pl.ds(0, W * D)],
                    out.at[pl.ds((base_c + t * K) * percol, W * D)], so[0])
            for b in range(2):
                wait_out(out, b) if (nblk % 2 == 0 or b == 1) else None
            if nblk % 2:
                wait_out(out, 0)

            @pl.when(wid < nextra)
            def _(tbl=tbl, out=out):
                c0 = base_c + ncols_lo
                pltpu.sync_copy(tbl.at[:, pl.ds(c0 * _C, _C)],
                                binb[0].at[:, pl.ds(0, _C)])
                transpose_strip(0, _C)
                pltpu.sync_copy(bout[0].at[pl.ds(0, percol)],
                                out.at[pl.ds(c0 * percol, percol)])

        for w_own, tail, out in ((nw - 2, tail1, o1), (nw - 1, tail2, o2)):
            @pl.when(wid == w_own)
            def _(tail=tail, out=out):
                for k in range(tail_rows // 8):
                    pltpu.sync_copy(tail.at[pl.ds(8 * k, 8), :], tbuf)
                    for r in range(8):
                        for f0 in (0, 16):
                            bout0[pl.ds(r * D + f0, 16)] = tbuf[r, pl.ds(f0, 16)]
                    pltpu.sync_copy(
                        bout0.at[pl.ds(0, 8 * D)],
                        out.at[pl.ds((full_cols * _C + 8 * k) * D, 8 * D)])

    return pl.kernel(
        body,
        mesh=mesh,
        out_type=(jax.ShapeDtypeStruct((V * D,), jnp.float32),
                  jax.ShapeDtypeStruct((V * D,), jnp.float32)),
        scratch_types=[
            pltpu.VMEM((D, P), jnp.float32),
            pltpu.VMEM((D, P), jnp.float32),
            pltpu.VMEM((W * D,), jnp.float32),
            pltpu.VMEM((W * D,), jnp.float32),
            pltpu.VMEM((8, _C), jnp.float32),
        ] + [pltpu.SemaphoreType.DMA] * 4,
        compiler_params=pltpu.CompilerParams(needs_layout_passes=False),
    )


def kernel(input, another_input, table1, table2):
    B, H = input.shape
    V, D = table1.shape
    N = B * H
    full_cols = V // _C
    i1 = input.T.reshape(N).astype(jnp.int32)
    i2 = another_input.T.reshape(N).astype(jnp.int32)
    tail1 = jnp.pad(table1[full_cols * _C:, :], ((0, 0), (0, _C - D)))
    tail2 = jnp.pad(table2[full_cols * _C:, :], ((0, 0), (0, _C - D)))
    o1, o2 = _build_transpose(V, D)(table1.T, table2.T, tail1, tail2)
    t1 = o1.reshape(V, D)
    t2 = o2.reshape(V, D)
    flat = _build(N, D, B)(t1, i1, t2, i2)
    out5 = flat.reshape(H, D // 8, B // 128, 8, 128)
    return out5.transpose(2, 4, 0, 1, 3).reshape(B, H, D)


# speedup vs baseline: 1.5899x; 1.5567x over previous
import functools

import jax
import jax.numpy as jnp
from jax import lax
from jax.experimental import pallas as pl
from jax.experimental.pallas import tpu as pltpu
from jax.experimental.pallas import tpu_sc as plsc

_C = 128
_NBUF = 4
_OBUF = 2


@functools.lru_cache(maxsize=None)
def _build(N, D, HB):
    info = plsc.get_sparse_core_info()
    nw = info.num_cores * info.num_subcores
    nper = N // nw
    steps = nper // _C
    outer = steps // _NBUF
    nfa = D // 8
    bblocks = HB // _C
    mesh = plsc.VectorSubcoreMesh(core_axis_name="c", subcore_axis_name="s")

    def body(t1, i1, t2, i2, out, i1v, i2v, r1, r2, ob, *sems):
        sg1 = sems[:_NBUF]
        sg2 = sems[_NBUF:2 * _NBUF]
        so = sems[2 * _NBUF:]
        wid = lax.axis_index("s") * info.num_cores + lax.axis_index("c")
        base = wid * nper
        iota = jax.lax.iota(jnp.int32, 16)
        pltpu.sync_copy(i1.at[pl.ds(base, nper)], i1v)
        pltpu.sync_copy(i2.at[pl.ds(base, nper)], i2v)

        def issue(g, b):
            o = g * _C
            pltpu.async_copy(t1.at[i1v.at[pl.ds(o, _C)]], r1.at[b], sg1[b])
            pltpu.async_copy(t2.at[i2v.at[pl.ds(o, _C)]], r2.at[b], sg2[b])

        def wait_gather(b):
            pltpu.make_async_copy(
                t1.at[i1v.at[pl.ds(0, _C)]], r1.at[b], sg1[b]).wait()
            pltpu.make_async_copy(
                t2.at[i2v.at[pl.ds(0, _C)]], r2.at[b], sg2[b]).wait()

        def wait_scatter(b2):
            for fa in range(nfa):
                pltpu.make_async_copy(
                    ob.at[b2, pl.ds(0, 8), pl.ds(0, _C)],
                    out.at[pl.ds(0, 8), :], so[b2]).wait()

        for b in range(_NBUF):
            issue(b, b)

        def outer_step(g2, carry):
            for b in range(_NBUF):
                g = g2 * _NBUF + b
                s = wid * steps + g
                h = s // bblocks
                ba = s % bblocks
                wait_gather(b)
                b2 = b % _OBUF
                if b < _OBUF:
                    @pl.when(g2 > 0)
                    def _():
                        wait_scatter(b2)
                else:
                    wait_scatter(b2)

                def tr_body(bb, c):
                    bs = jnp.full((16,), bb, jnp.int32)
                    for f0 in (0, 16):
                        sl = pl.ds(f0, 16)
                        v = r1[b, bb, sl] + r2[b, bb, sl]
                        plsc.store_scatter(ob.at[b2], [f0 + iota, bs], v)
                    return c

                lax.fori_loop(0, _C, tr_body, 0, unroll=8)

                @pl.when(g2 < outer - 1)
                def _():
                    issue(g + _NBUF, b)

                for fa in range(nfa):
                    off8 = (h * nfa + fa) * (bblocks * 8) + ba * 8
                    pltpu.async_copy(ob.at[b2, pl.ds(fa * 8, 8), pl.ds(0, _C)],
                                     out.at[pl.ds(off8, 8), :], so[b2])
            return carry

        lax.fori_loop(0, outer, outer_step, 0)
        for b2 in range(_OBUF):
            wait_scatter(b2)

    return pl.kernel(
        body,
        mesh=mesh,
        out_type=jax.ShapeDtypeStruct((N * D // _C, _C), jnp.float32),
        scratch_types=[
            pltpu.VMEM((nper,), jnp.int32),
            pltpu.VMEM((nper,), jnp.int32),
            pltpu.VMEM((_NBUF, _C, D), jnp.float32),
            pltpu.VMEM((_NBUF, _C, D), jnp.float32),
            pltpu.VMEM((_OBUF, nfa * 8, _C + 1), jnp.float32),
        ] + [pltpu.SemaphoreType.DMA] * (2 * _NBUF + _OBUF),
        compiler_params=pltpu.CompilerParams(use_tc_tiling_on_sc=False,
                                             needs_layout_passes=False),
    )


@functools.lru_cache(maxsize=None)
def _build_transpose(V, D):
    info = plsc.get_sparse_core_info()
    ncore, nsub = info.num_cores, info.num_subcores
    nw = ncore * nsub
    full_cols = V // _C
    tail_rows = V - full_cols * _C
    percol = _C * D
    ncols_lo = full_cols // nw
    nextra = full_cols - ncols_lo * nw
    K = 4
    W = K * _C
    P = W + 5
    nblk = ncols_lo // K
    mesh = plsc.VectorSubcoreMesh(core_axis_name="c", subcore_axis_name="s")

    def body(t1t, t2t, tail1, tail2, o1, o2, bin0, bin1, bout0, bout1, tbuf,
             *sems):
        binb = (bin0, bin1)
        bout = (bout0, bout1)
        si = sems[:2]
        so = sems[2:]
        wid = lax.axis_index("s") * ncore + lax.axis_index("c")
        base_c = wid * ncols_lo + jnp.minimum(wid, nextra)
        iota = jax.lax.iota(jnp.int32, 16)

        def issue(tbl, t, b, width=W):
            pltpu.async_copy(
                tbl.at[:, pl.ds((base_c + t * K) * _C, width)],
                binb[b].at[:, pl.ds(0, width)], si[b])

        def wait_in(tbl, b, width=W):
            pltpu.make_async_copy(
                tbl.at[:, pl.ds(0, width)], binb[b].at[:, pl.ds(0, width)],
                si[b]).wait()

        def wait_out(out, b, words=W * D):
            pltpu.make_async_copy(
                bout[b].at[pl.ds(0, words)], out.at[pl.ds(0, words)],
                so[b]).wait()

        def transpose_strip(b, width):
            def row_body(r, c):
                rs = jnp.full((16,), r, jnp.int32)
                for f0 in (0, 16):
                    v = plsc.load_gather(binb[b], [f0 + iota, rs])
                    bout[b][pl.ds(r * D + f0, 16)] = v
                return c

            lax.fori_loop(0, width, row_body, 0, unroll=8)

        for tbl, out in ((t1t, o1), (t2t, o2)):
            def issue2(t, b, tbl=tbl):
                issue(tbl, t, b)

            for b in range(2):
                issue2(b, b)

            def step(t2_, carry, tbl=tbl, out=out):
                for b in range(2):
                    t = t2_ * 2 + b
                    wait_in(tbl, b)
                    @pl.when(t2_ > 0)
                    def _():
                        wait_out(out, b)
                    transpose_strip(b, W)
                    pltpu.async_copy(
                        bout[b].at[pl.ds(0, W * D)],
                        out.at[pl.ds((base_c + t * K) * percol, W * D)], so[b])
                    @pl.when(t + 2 < nblk)
                    def _():
                        issue2(t + 2, b)
                return carry

            lax.fori_loop(0, nblk // 2, step, 0)

            if nblk % 2:
                t = nblk - 1
                wait_in(tbl, 0)
                wait_out(out, 0)
                transpose_strip(0, W)
                pltpu.async_copy(
                    bout[0].at[pl.ds(0, W * D)],
                    out.at[pl.ds((base_c + t * K) * percol, W * D)], so[0])
            for b in range(2):
                wait_out(out, b) if (nblk % 2 == 0 or b == 1) else None
            if nblk % 2:
                wait_out(out, 0)

            @pl.when(wid < nextra)
            def _(tbl=tbl, out=out):
                c0 = base_c + ncols_lo
                pltpu.sync_copy(tbl.at[:, pl.ds(c0 * _C, _C)],
                                binb[0].at[:, pl.ds(0, _C)])
                transpose_strip(0, _C)
                pltpu.sync_copy(bout[0].at[pl.ds(0, percol)],
                                out.at[pl.ds(c0 * percol, percol)])

        for w_own, tail, out in ((nw - 2, tail1, o1), (nw - 1, tail2, o2)):
            @pl.when(wid == w_own)
            def _(tail=tail, out=out):
                for k in range(tail_rows // 8):
                    pltpu.sync_copy(tail.at[pl.ds(8 * k, 8), :], tbuf)
                    for r in range(8):
                        for f0 in (0, 16):
                            bout0[pl.ds(r * D + f0, 16)] = tbuf[r, pl.ds(f0, 16)]
                    pltpu.sync_copy(
                        bout0.at[pl.ds(0, 8 * D)],
                        out.at[pl.ds((full_cols * _C + 8 * k) * D, 8 * D)])

    return pl.kernel(
        body,
        mesh=mesh,
        out_type=(jax.ShapeDtypeStruct((V * D,), jnp.float32),
                  jax.ShapeDtypeStruct((V * D,), jnp.float32)),
        scratch_types=[
            pltpu.VMEM((D, P), jnp.float32),
            pltpu.VMEM((D, P), jnp.float32),
            pltpu.VMEM((W * D,), jnp.float32),
            pltpu.VMEM((W * D,), jnp.float32),
            pltpu.VMEM((8, _C), jnp.float32),
        ] + [pltpu.SemaphoreType.DMA] * 4,
        compiler_params=pltpu.CompilerParams(needs_layout_passes=False),
    )


def kernel(input, another_input, table1, table2):
    B, H = input.shape
    V, D = table1.shape
    N = B * H
    full_cols = V // _C
    i1 = input.T.reshape(N).astype(jnp.int32)
    i2 = another_input.T.reshape(N).astype(jnp.int32)
    flat = _build(N, D, B)(table1, i1, table2, i2)
    out5 = flat.reshape(H, D // 8, B // 128, 8, 128)
    return out5.transpose(2, 4, 0, 1, 3).reshape(B, H, D)
